# baseline (speedup 1.0000x reference)
import jax
import jax.numpy as jnp
from jax import lax
from jax.experimental import pallas as pl
from jax.experimental.pallas import tpu as pltpu
from jax.experimental.pallas import tpu_sc as plsc

NC = 2
NS = 16
LANES = 16
NW = NC * NS

BATCH = 4
MAXLEN = 8192
DIM = 768

SEQ_PER_W = MAXLEN // NW
CHUNK_SEQ = 16
CHUNK_WORDS = CHUNK_SEQ * DIM
N_CHUNKS = SEQ_PER_W // CHUNK_SEQ
UNROLL = 8


def _sc_add(x_hbm, pos_hbm, out_hbm,
            x00, x01, x02, x03, x10, x11, x12, x13,
            pb0, pb1,
            sx0, sx1, sp0, sp1, so0, so1):
    wid = lax.axis_index("s") * NC + lax.axis_index("c")
    seq_base = wid * SEQ_PER_W

    xbufs = ((x00, x01, x02, x03), (x10, x11, x12, x13))
    pbufs = (pb0, pb1)
    sxs, sps, sos = (sx0, sx1), (sp0, sp1), (so0, so1)

    def x_off(b, c):
        return (b * MAXLEN + seq_base + c * CHUNK_SEQ) * DIM

    def load(c):
        slot = c % 2
        hs = [pltpu.async_copy(
            pos_hbm.at[pl.ds((seq_base + c * CHUNK_SEQ) * DIM, CHUNK_WORDS)],
            pbufs[slot], sps[slot])]
        for b in range(BATCH):
            hs.append(pltpu.async_copy(
                x_hbm.at[pl.ds(x_off(b, c), CHUNK_WORDS)],
                xbufs[slot][b], sxs[slot]))
        return hs

    def store(c):
        slot = c % 2
        return [pltpu.async_copy(
            xbufs[slot][b], out_hbm.at[pl.ds(x_off(b, c), CHUNK_WORDS)],
            sos[slot]) for b in range(BATCH)]

    def compute(slot):
        pb = pbufs[slot]
        xbs = xbufs[slot]

        @plsc.parallel_loop(0, CHUNK_WORDS // LANES, unroll=UNROLL)
        def vec_body(i):
            s = pl.ds(i * LANES, LANES)
            p = pb[s]
            for b in range(BATCH):
                xbs[b][s] = xbs[b][s] + p

    loads = [None, None]
    stores = [None, None]
    loads[0] = load(0)
    for c in range(N_CHUNKS):
        slot = c % 2
        if c + 1 < N_CHUNKS:
            nslot = (c + 1) % 2
            if stores[nslot] is not None:
                for h in stores[nslot]:
                    h.wait()
                stores[nslot] = None
            loads[nslot] = load(c + 1)
        for h in loads[slot]:
            h.wait()
        compute(slot)
        stores[slot] = store(c)
    for hs in stores:
        if hs is not None:
            for h in hs:
                h.wait()


_sc_kernel = pl.kernel(
    _sc_add,
    out_type=jax.ShapeDtypeStruct((BATCH * MAXLEN * DIM,), jnp.float32),
    mesh=plsc.VectorSubcoreMesh(core_axis_name="c", subcore_axis_name="s"),
    scratch_types=(
        [pltpu.VMEM((CHUNK_WORDS,), jnp.float32) for _ in range(8)]
        + [pltpu.VMEM((CHUNK_WORDS,), jnp.float32) for _ in range(2)]
        + [pltpu.SemaphoreType.DMA for _ in range(6)]
    ),
)


def kernel(x, pos_table):
    out = _sc_kernel(x.reshape(-1), pos_table.reshape(-1))
    return out.reshape(x.shape)
